# baseline (speedup 1.0000x reference)
import functools

import jax
import jax.numpy as jnp
from jax import lax
from jax.experimental import pallas as pl
from jax.experimental.pallas import tpu as pltpu
from jax.experimental.pallas import tpu_sc as plsc

_B, _T, _L, _D, _H = 4, 2048, 15, 256, 128
_TL = _T * _L
_K = 410
_KP = 416
_PRUNE_RATIO = 0.2
_TT = 128
_RB = _TT * _L
_NT = _T // _TT
_CH = 1920
_NCH = _TL // _CH
_M15 = 69906
_SH15 = 20


def _score_body(x_ref, m_ref, gb_ref, gl_ref, w1_ref, b1_ref, w2_ref, b2_ref,
                w3_ref, b3_ref, ps_ref, loss_ref):
    b = pl.program_id(0)
    i = pl.program_id(1)
    x = x_ref[...]
    h = jnp.maximum(jnp.dot(x, w1_ref[...]) + b1_ref[...], 0.0)
    h = jnp.maximum(jnp.dot(h, w2_ref[...]) + b2_ref[...], 0.0)
    s = jnp.dot(h, w3_ref[...]) + b3_ref[...]
    m = m_ref[...]
    ps = s - (1.0 - m) * 10000.0
    ps_ref[...] = ps

    r = lax.broadcasted_iota(jnp.int32, (_RB, 1), 0)
    q = lax.shift_right_arithmetic(r * _M15, _SH15)
    t_row = i * _TT + q
    l_row = r - q * _L
    gb = gb_ref[...]
    gl = gl_ref[...]
    validg = (gl >= 0) & (gl < _L) & (gb >= 0) & (gb < _T)
    hit = (t_row == gb) & (l_row == gl) & validg
    tgt = jnp.any(hit, axis=1, keepdims=True).astype(jnp.float32)

    bce = jnp.maximum(ps, 0.0) - ps * tgt + jnp.log1p(jnp.exp(-jnp.abs(ps)))
    part = jnp.sum(bce * m)

    @pl.when((b == 0) & (i == 0))
    def _():
        loss_ref[...] = jnp.zeros((1, 1), jnp.float32)

    loss_ref[...] += part


def _topk_body(ps_ref, sl_ref, oidx_ref, osc_ref, ob_ref, oe_ref,
               sq_ref, tri_ref):
    ps = ps_ref[...]
    bits = lax.bitcast_convert_type(ps, jnp.int32)
    key = jnp.where(bits >= 0, bits, bits ^ 0x7FFFFFFF)

    def cnt_ge(c):
        return jnp.sum(jnp.where(key >= c, 1, 0))

    def step(j, mcur):
        bit = 30 - j
        cand = mcur + lax.shift_left(jnp.int32(1), bit)
        return jnp.where(cnt_ge(cand) >= _K, cand, mcur)

    m0 = jnp.where(cnt_ge(jnp.int32(0)) >= _K,
                   jnp.int32(0), jnp.int32(-2147483648))
    mth = lax.fori_loop(0, 31, step, m0)
    need = _K - jnp.sum(jnp.where(key > mth, 1, 0))

    eqm = key == mth
    pos = lax.broadcasted_iota(jnp.int32, (1, _TL), 1)

    def step2(j, x):
        bit = 14 - j
        xc = x + lax.shift_left(jnp.int32(1), bit)
        cnt = jnp.sum(jnp.where(eqm & (pos < xc), 1, 0))
        return jnp.where(cnt < need, xc, x)

    istar = lax.fori_loop(0, 15, step2, jnp.int32(0))

    keep = (key > mth) | (eqm & (pos <= istar))
    k01 = jnp.where(keep, 1, 0)

    gcum = k01
    sh = 1
    while sh < _TL:
        shifted = jnp.concatenate(
            [jnp.zeros((1, sh), jnp.int32), gcum[:, :_TL - sh]], axis=1)
        gcum = gcum + shifted
        sh *= 2

    kk1 = lax.broadcasted_iota(jnp.int32, (_KP, 1), 0) + 1
    accI = jnp.zeros((_KP, 1), jnp.float32)
    accS = jnp.zeros((_KP, 1), jnp.float32)
    for c in range(_NCH):
        lo = c * _CH
        g_c = gcum[:, lo:lo + _CH]
        f_c = k01[:, lo:lo + _CH]
        p_c = ps[:, lo:lo + _CH]
        oh = jnp.where((g_c == kk1) & (f_c == 1), 1.0, 0.0)
        e_c = (lax.broadcasted_iota(jnp.int32, (1, _CH), 1) + lo
               ).astype(jnp.float32)
        accI = accI + jnp.sum(oh * e_c, axis=1, keepdims=True)
        accS = accS + jnp.sum(oh * p_c, axis=1, keepdims=True)

    idx_i = accI.astype(jnp.int32)
    qk = lax.shift_right_arithmetic(idx_i * _M15, _SH15)
    oidx_ref[...] = idx_i
    osc_ref[...] = accS
    ob_ref[...] = qk
    oe_ref[...] = idx_i - qk * (_L - 1)

    sl = sl_ref[...].astype(jnp.float32)
    spl = jnp.minimum(jnp.ceil(_PRUNE_RATIO * sl).astype(jnp.int32), _K)
    ri = lax.broadcasted_iota(jnp.int32, (_K, _K), 0)
    rj = lax.broadcasted_iota(jnp.int32, (_K, _K), 1)
    vi = ri < spl
    vj = rj < spl
    sq = jnp.where(vi & vj, 1.0, 0.0)
    sq_ref[...] = sq
    tri_ref[...] = sq * jnp.where(rj <= ri, 1.0, 0.0)


_CHUNKS = ((0, 104), (104, 104), (208, 104), (312, 104))


def _sc_body(idx_hbm, svf_hbm, fv_hbm, idx_v, idxb_v, rows_v, sem):
    cc = lax.axis_index("c")
    ss = lax.axis_index("s")
    wid = ss * 2 + cc

    @pl.when(wid < _B)
    def _():
        b = wid
        pltpu.sync_copy(idx_hbm.at[b], idx_v)
        for j in range(_KP // 16):
            iv = idx_v[pl.ds(j * 16, 16)]
            idxb_v[pl.ds(j * 16, 16)] = iv + b * _TL

        for (s0, n) in _CHUNKS:
            pltpu.async_copy(svf_hbm.at[idxb_v.at[pl.ds(s0, n)]],
                             rows_v.at[pl.ds(0, n)], sem).wait()
            pltpu.sync_copy(rows_v.at[pl.ds(0, n)],
                            fv_hbm.at[pl.ds(b * _KP + s0, n)])


def _run_score(sv2, m2, gbm, glm, W1, b1, W2, b2, W3, b3):
    return pl.pallas_call(
        _score_body,
        grid=(_B, _NT),
        in_specs=[
            pl.BlockSpec((None, _RB, _D), lambda b, i: (b, i, 0)),
            pl.BlockSpec((None, _RB, 1), lambda b, i: (b, i, 0)),
            pl.BlockSpec((None, 1, gbm.shape[2]), lambda b, i: (b, 0, 0)),
            pl.BlockSpec((None, 1, glm.shape[2]), lambda b, i: (b, 0, 0)),
            pl.BlockSpec((_D, _H), lambda b, i: (0, 0)),
            pl.BlockSpec((1, _H), lambda b, i: (0, 0)),
            pl.BlockSpec((_H, _H), lambda b, i: (0, 0)),
            pl.BlockSpec((1, _H), lambda b, i: (0, 0)),
            pl.BlockSpec((_H, 1), lambda b, i: (0, 0)),
            pl.BlockSpec((1, 1), lambda b, i: (0, 0)),
        ],
        out_specs=[
            pl.BlockSpec((None, _RB, 1), lambda b, i: (b, i, 0)),
            pl.BlockSpec((1, 1), lambda b, i: (0, 0)),
        ],
        out_shape=[
            jax.ShapeDtypeStruct((_B, _TL, 1), jnp.float32),
            jax.ShapeDtypeStruct((1, 1), jnp.float32),
        ],
    )(sv2, m2, gbm, glm, W1, b1, W2, b2, W3, b3)


def _run_topk(ps3, sl3):
    return pl.pallas_call(
        _topk_body,
        grid=(_B,),
        in_specs=[
            pl.BlockSpec((None, 1, _TL), lambda b: (b, 0, 0)),
            pl.BlockSpec((None, 1, 1), lambda b: (b, 0, 0)),
        ],
        out_specs=[
            pl.BlockSpec((None, _KP, 1), lambda b: (b, 0, 0)),
            pl.BlockSpec((None, _KP, 1), lambda b: (b, 0, 0)),
            pl.BlockSpec((None, _KP, 1), lambda b: (b, 0, 0)),
            pl.BlockSpec((None, _KP, 1), lambda b: (b, 0, 0)),
            pl.BlockSpec((None, _K, _K), lambda b: (b, 0, 0)),
            pl.BlockSpec((None, _K, _K), lambda b: (b, 0, 0)),
        ],
        out_shape=[
            jax.ShapeDtypeStruct((_B, _KP, 1), jnp.int32),
            jax.ShapeDtypeStruct((_B, _KP, 1), jnp.float32),
            jax.ShapeDtypeStruct((_B, _KP, 1), jnp.int32),
            jax.ShapeDtypeStruct((_B, _KP, 1), jnp.int32),
            jax.ShapeDtypeStruct((_B, _K, _K), jnp.float32),
            jax.ShapeDtypeStruct((_B, _K, _K), jnp.float32),
        ],
    )(ps3, sl3)


@functools.cache
def _make_prune():
    return pl.kernel(
        _sc_body,
        mesh=plsc.VectorSubcoreMesh(core_axis_name="c", subcore_axis_name="s"),
        out_type=[
            jax.ShapeDtypeStruct((_B * _KP, _D), jnp.float32),
        ],
        scratch_types=[
            pltpu.VMEM((_KP,), jnp.int32),
            pltpu.VMEM((_KP,), jnp.int32),
            pltpu.VMEM((104, _D), jnp.float32),
            pltpu.SemaphoreType.DMA,
        ],
    )


def _run_prune(idx2, svf):
    return _make_prune()(idx2, svf)


def kernel(span_vecs, span_mask, W1, b1, W2, b2, W3, b3,
           span_begin, span_end, sequence_lengths, gold_spans):
    B, T, L, D = span_vecs.shape
    H = W1.shape[1]
    G = gold_spans.shape[1]
    sv2 = span_vecs.reshape(B, T * L, D)
    m2 = span_mask.reshape(B, T * L, 1)
    gbm = gold_spans[..., 0].reshape(B, 1, G)
    glm = (gold_spans[..., 1] - gold_spans[..., 0]).reshape(B, 1, G)

    ps, loss = _run_score(sv2, m2, gbm, glm, W1, b1.reshape(1, H),
                          W2, b2.reshape(1, H), W3, b3.reshape(1, 1))
    ps3 = ps.reshape(B, 1, T * L)
    oidx, osc, ob, oe, sq, tri = _run_topk(
        ps3, sequence_lengths.reshape(B, 1, 1))
    idx2 = oidx.reshape(B, _KP)
    fv, = _run_prune(idx2, sv2.reshape(B * T * L, D))

    obj = loss[0, 0]
    prune_scores = ps.reshape(B, T, L, 1)
    f_vecs = fv.reshape(B, _KP, D)[:, :_K]
    f_scores = osc[:, :_K]
    f_begin = ob[:, :_K, 0]
    f_end = oe[:, :_K, 0]
    idx = idx2[:, :_K]
    return (obj, prune_scores, f_vecs, f_scores, f_begin, f_end, sq, tri, idx)


# speedup vs baseline: 1.0889x; 1.0889x over previous
import functools

import jax
import jax.numpy as jnp
from jax import lax
from jax.experimental import pallas as pl
from jax.experimental.pallas import tpu as pltpu
from jax.experimental.pallas import tpu_sc as plsc

_B, _T, _L, _D, _H = 4, 2048, 15, 256, 128
_TL = _T * _L
_K = 410
_KP = 416
_PRUNE_RATIO = 0.2
_TT = 128
_RB = _TT * _L
_NT = _T // _TT
_CH = 1920
_NCH = _TL // _CH
_M15 = 69906
_SH15 = 20


def _score_body(x_ref, m_ref, gb_ref, gl_ref, w1_ref, b1_ref, w2_ref, b2_ref,
                w3_ref, b3_ref, ps_ref, loss_ref):
    b = pl.program_id(0)
    i = pl.program_id(1)
    x = x_ref[...]
    h = jnp.maximum(jnp.dot(x, w1_ref[...]) + b1_ref[...], 0.0)
    h = jnp.maximum(jnp.dot(h, w2_ref[...]) + b2_ref[...], 0.0)
    s = jnp.dot(h, w3_ref[...]) + b3_ref[...]
    sr = s.reshape(1, _RB)
    m = m_ref[...]
    ps = sr - (1.0 - m) * 10000.0
    ps_ref[...] = ps

    r = lax.broadcasted_iota(jnp.int32, (1, _RB), 1)
    q = lax.shift_right_arithmetic(r * _M15, _SH15)
    t_row = i * _TT + q
    l_row = r - q * _L
    gb = gb_ref[...]
    gl = gl_ref[...]
    validg = (gl >= 0) & (gl < _L) & (gb >= 0) & (gb < _T)
    hit = (t_row == gb) & (l_row == gl) & validg
    tgt = jnp.any(hit, axis=0, keepdims=True).astype(jnp.float32)

    bce = jnp.maximum(ps, 0.0) - ps * tgt + jnp.log1p(jnp.exp(-jnp.abs(ps)))
    part = jnp.sum(bce * m)

    @pl.when((b == 0) & (i == 0))
    def _():
        loss_ref[...] = jnp.zeros((1, 1), jnp.float32)

    loss_ref[...] += part


def _topk_body(ps_ref, sl_ref, oidx_ref, osc_ref, ob_ref, oe_ref,
               sq_ref, tri_ref):
    ps = ps_ref[...]
    bits = lax.bitcast_convert_type(ps, jnp.int32)
    key = jnp.where(bits >= 0, bits, bits ^ 0x7FFFFFFF)

    def cnt_ge(c):
        return jnp.sum(jnp.where(key >= c, 1, 0))

    def step(j, mcur):
        bit = 30 - j
        cand = mcur + lax.shift_left(jnp.int32(1), bit)
        return jnp.where(cnt_ge(cand) >= _K, cand, mcur)

    m0 = jnp.where(cnt_ge(jnp.int32(0)) >= _K,
                   jnp.int32(0), jnp.int32(-2147483648))
    mth = lax.fori_loop(0, 31, step, m0)
    need = _K - jnp.sum(jnp.where(key > mth, 1, 0))

    eqm = key == mth
    pos = (lax.broadcasted_iota(jnp.int32, (_NT, _RB), 0) * _RB
           + lax.broadcasted_iota(jnp.int32, (_NT, _RB), 1))

    def step2(j, x):
        bit = 14 - j
        xc = x + lax.shift_left(jnp.int32(1), bit)
        cnt = jnp.sum(jnp.where(eqm & (pos < xc), 1, 0))
        return jnp.where(cnt < need, xc, x)

    istar = lax.fori_loop(0, 15, step2, jnp.int32(0))

    keep = (key > mth) | (eqm & (pos <= istar))
    k01 = jnp.where(keep, 1, 0)

    rcum = k01
    sh = 1
    while sh < _RB:
        shifted = jnp.concatenate(
            [jnp.zeros((_NT, sh), jnp.int32), rcum[:, :_RB - sh]], axis=1)
        rcum = rcum + shifted
        sh *= 2
    rtot = rcum[:, _RB - 1:_RB]
    roff = jnp.zeros((_NT, 1), jnp.int32)
    sh = 1
    acc = rtot
    while sh < _NT:
        shifted = jnp.concatenate(
            [jnp.zeros((sh, 1), jnp.int32), acc[:_NT - sh, :]], axis=0)
        acc = acc + shifted
        sh *= 2
    roff = jnp.concatenate(
        [jnp.zeros((1, 1), jnp.int32), acc[:_NT - 1, :]], axis=0)
    gcum = rcum + roff

    kk1 = lax.broadcasted_iota(jnp.int32, (_KP, 1), 0) + 1
    accI = jnp.zeros((_KP, 1), jnp.float32)
    accS = jnp.zeros((_KP, 1), jnp.float32)
    for c in range(_NT):
        g_c = gcum[c:c + 1, :]
        f_c = k01[c:c + 1, :]
        p_c = ps[c:c + 1, :]
        oh = jnp.where((g_c == kk1) & (f_c == 1), 1.0, 0.0)
        e_c = (lax.broadcasted_iota(jnp.int32, (1, _RB), 1) + c * _RB
               ).astype(jnp.float32)
        accI = accI + jnp.sum(oh * e_c, axis=1, keepdims=True)
        accS = accS + jnp.sum(oh * p_c, axis=1, keepdims=True)

    idx_i = accI.astype(jnp.int32)
    qk = lax.shift_right_arithmetic(idx_i * _M15, _SH15)
    oidx_ref[...] = idx_i.reshape(1, _KP)
    osc_ref[...] = accS.reshape(1, _KP)
    ob_ref[...] = qk.reshape(1, _KP)
    oe_ref[...] = (idx_i - qk * (_L - 1)).reshape(1, _KP)

    sl = sl_ref[...].astype(jnp.float32)
    spl = jnp.minimum(jnp.ceil(_PRUNE_RATIO * sl).astype(jnp.int32), _K)
    ri = lax.broadcasted_iota(jnp.int32, (_K, _K), 0)
    rj = lax.broadcasted_iota(jnp.int32, (_K, _K), 1)
    vi = ri < spl
    vj = rj < spl
    sq = jnp.where(vi & vj, 1.0, 0.0)
    sq_ref[...] = sq
    tri_ref[...] = sq * jnp.where(rj <= ri, 1.0, 0.0)


_CHUNKS = ((0, 104), (104, 104), (208, 104), (312, 104))


def _sc_body(idx_hbm, svf_hbm, fv_hbm, idx_v, idxb_v, rows_v, sem):
    cc = lax.axis_index("c")
    ss = lax.axis_index("s")
    wid = ss * 2 + cc

    @pl.when(wid < _B)
    def _():
        b = wid
        pltpu.sync_copy(idx_hbm.at[b], idx_v)
        for j in range(_KP // 16):
            iv = idx_v[pl.ds(j * 16, 16)]
            idxb_v[pl.ds(j * 16, 16)] = iv + b * _TL

        for (s0, n) in _CHUNKS:
            pltpu.async_copy(svf_hbm.at[idxb_v.at[pl.ds(s0, n)]],
                             rows_v.at[pl.ds(0, n)], sem).wait()
            pltpu.sync_copy(rows_v.at[pl.ds(0, n)],
                            fv_hbm.at[pl.ds(b * _KP + s0, n)])


def _run_score(sv2, m2, gbm, glm, W1, b1, W2, b2, W3, b3):
    return pl.pallas_call(
        _score_body,
        grid=(_B, _NT),
        in_specs=[
            pl.BlockSpec((None, _RB, _D), lambda b, i: (b, i, 0)),
            pl.BlockSpec((None, None, 1, _RB), lambda b, i: (b, i, 0, 0)),
            pl.BlockSpec((None, gbm.shape[1], 1), lambda b, i: (b, 0, 0)),
            pl.BlockSpec((None, glm.shape[1], 1), lambda b, i: (b, 0, 0)),
            pl.BlockSpec((_D, _H), lambda b, i: (0, 0)),
            pl.BlockSpec((1, _H), lambda b, i: (0, 0)),
            pl.BlockSpec((_H, _H), lambda b, i: (0, 0)),
            pl.BlockSpec((1, _H), lambda b, i: (0, 0)),
            pl.BlockSpec((_H, 1), lambda b, i: (0, 0)),
            pl.BlockSpec((1, 1), lambda b, i: (0, 0)),
        ],
        out_specs=[
            pl.BlockSpec((None, None, 1, _RB), lambda b, i: (b, i, 0, 0)),
            pl.BlockSpec((1, 1), lambda b, i: (0, 0)),
        ],
        out_shape=[
            jax.ShapeDtypeStruct((_B, _NT, 1, _RB), jnp.float32),
            jax.ShapeDtypeStruct((1, 1), jnp.float32),
        ],
    )(sv2, m2, gbm, glm, W1, b1, W2, b2, W3, b3)


def _run_topk(ps3, sl3):
    return pl.pallas_call(
        _topk_body,
        grid=(_B,),
        in_specs=[
            pl.BlockSpec((None, _NT, _RB), lambda b: (b, 0, 0)),
            pl.BlockSpec((None, 1, 1), lambda b: (b, 0, 0)),
        ],
        out_specs=[
            pl.BlockSpec((None, 1, _KP), lambda b: (b, 0, 0)),
            pl.BlockSpec((None, 1, _KP), lambda b: (b, 0, 0)),
            pl.BlockSpec((None, 1, _KP), lambda b: (b, 0, 0)),
            pl.BlockSpec((None, 1, _KP), lambda b: (b, 0, 0)),
            pl.BlockSpec((None, _K, _K), lambda b: (b, 0, 0)),
            pl.BlockSpec((None, _K, _K), lambda b: (b, 0, 0)),
        ],
        out_shape=[
            jax.ShapeDtypeStruct((_B, 1, _KP), jnp.int32),
            jax.ShapeDtypeStruct((_B, 1, _KP), jnp.float32),
            jax.ShapeDtypeStruct((_B, 1, _KP), jnp.int32),
            jax.ShapeDtypeStruct((_B, 1, _KP), jnp.int32),
            jax.ShapeDtypeStruct((_B, _K, _K), jnp.float32),
            jax.ShapeDtypeStruct((_B, _K, _K), jnp.float32),
        ],
    )(ps3, sl3)


@functools.cache
def _make_prune():
    return pl.kernel(
        _sc_body,
        mesh=plsc.VectorSubcoreMesh(core_axis_name="c", subcore_axis_name="s"),
        out_type=[
            jax.ShapeDtypeStruct((_B * _KP, _D), jnp.float32),
        ],
        scratch_types=[
            pltpu.VMEM((_KP,), jnp.int32),
            pltpu.VMEM((_KP,), jnp.int32),
            pltpu.VMEM((104, _D), jnp.float32),
            pltpu.SemaphoreType.DMA,
        ],
    )


def _run_prune(idx2, svf):
    return _make_prune()(idx2, svf)


def kernel(span_vecs, span_mask, W1, b1, W2, b2, W3, b3,
           span_begin, span_end, sequence_lengths, gold_spans):
    B, T, L, D = span_vecs.shape
    H = W1.shape[1]
    G = gold_spans.shape[1]
    sv2 = span_vecs.reshape(B, T * L, D)
    m2 = span_mask.reshape(B, _NT, 1, _RB)
    gbm = gold_spans[..., 0].reshape(B, G, 1)
    glm = (gold_spans[..., 1] - gold_spans[..., 0]).reshape(B, G, 1)

    ps, loss = _run_score(sv2, m2, gbm, glm, W1, b1.reshape(1, H),
                          W2, b2.reshape(1, H), W3, b3.reshape(1, 1))
    oidx, osc, ob, oe, sq, tri = _run_topk(
        ps.reshape(B, _NT, _RB), sequence_lengths.reshape(B, 1, 1))
    idx2 = oidx.reshape(B, _KP)
    fv, = _run_prune(idx2, sv2.reshape(B * T * L, D))

    obj = loss[0, 0]
    prune_scores = ps.reshape(B, T, L, 1)
    f_vecs = fv.reshape(B, _KP, D)[:, :_K]
    f_scores = osc.reshape(B, _KP)[:, :_K, None]
    f_begin = ob.reshape(B, _KP)[:, :_K]
    f_end = oe.reshape(B, _KP)[:, :_K]
    idx = idx2[:, :_K]
    return (obj, prune_scores, f_vecs, f_scores, f_begin, f_end, sq, tri, idx)


# speedup vs baseline: 1.2430x; 1.1415x over previous
import functools

import jax
import jax.numpy as jnp
from jax import lax
from jax.experimental import pallas as pl
from jax.experimental.pallas import tpu as pltpu
from jax.experimental.pallas import tpu_sc as plsc

_B, _T, _L, _D, _H = 4, 2048, 15, 256, 128
_TL = _T * _L
_K = 410
_KP = 416
_PRUNE_RATIO = 0.2
_TT = 128
_RB = _TT * _L
_NT = _T // _TT
_CH = 1920
_NCH = _TL // _CH
_M15 = 69906
_SH15 = 20


def _score_body(x_ref, sl_ref, gb_ref, gl_ref, w1_ref, b1_ref, w2_ref, b2_ref,
                w3_ref, b3_ref, ps_ref, loss_ref):
    b = pl.program_id(0)
    i = pl.program_id(1)
    x = x_ref[...]
    h = jnp.maximum(jnp.dot(x, w1_ref[...]) + b1_ref[...], 0.0)
    h = jnp.maximum(jnp.dot(h, w2_ref[...]) + b2_ref[...], 0.0)
    sr = lax.dot_general(w3_ref[...], h, (((1,), (1,)), ((), ())))
    sr = sr + b3_ref[0, 0]

    r = lax.broadcasted_iota(jnp.int32, (1, _RB), 1)
    q = lax.shift_right_arithmetic(r * _M15, _SH15)
    t_row = i * _TT + q
    l_row = r - q * _L
    slen = sl_ref[0, 0]
    m = jnp.where(t_row + l_row < slen, 1.0, 0.0)
    ps = sr - (1.0 - m) * 10000.0
    ps_ref[pl.ds(lax.rem(i, 8), 1), :] = ps

    gb = gb_ref[...]
    gl = gl_ref[...]
    validg = (gl >= 0) & (gl < _L) & (gb >= 0) & (gb < _T)
    hit = (t_row == gb) & (l_row == gl) & validg
    tgt = jnp.any(hit, axis=0, keepdims=True).astype(jnp.float32)

    bce = jnp.maximum(ps, 0.0) - ps * tgt + jnp.log(1.0 + jnp.exp(-jnp.abs(ps)))
    part = jnp.sum(bce * m)

    @pl.when((b == 0) & (i == 0))
    def _():
        loss_ref[...] = jnp.zeros((1, 1), jnp.float32)

    loss_ref[...] += part


def _topk_body(ps_ref, sl_ref, oidx_ref, osc_ref, ob_ref, oe_ref,
               sq_ref, tri_ref):
    ps = ps_ref[...]
    bits = lax.bitcast_convert_type(ps, jnp.int32)
    key = jnp.where(bits >= 0, bits, bits ^ 0x7FFFFFFF)

    def cnt_ge(c):
        return jnp.sum(jnp.where(key >= c, 1, 0))

    def step(j, mcur):
        bit = 30 - j
        cand = mcur + lax.shift_left(jnp.int32(1), bit)
        return jnp.where(cnt_ge(cand) >= _K, cand, mcur)

    m0 = jnp.where(cnt_ge(jnp.int32(0)) >= _K,
                   jnp.int32(0), jnp.int32(-2147483648))
    mth = lax.fori_loop(0, 31, step, m0)
    need = _K - jnp.sum(jnp.where(key > mth, 1, 0))

    eqm = key == mth
    pos = (lax.broadcasted_iota(jnp.int32, (_NT, _RB), 0) * _RB
           + lax.broadcasted_iota(jnp.int32, (_NT, _RB), 1))

    def step2(j, x):
        bit = 14 - j
        xc = x + lax.shift_left(jnp.int32(1), bit)
        cnt = jnp.sum(jnp.where(eqm & (pos < xc), 1, 0))
        return jnp.where(cnt < need, xc, x)

    istar = lax.fori_loop(0, 15, step2, jnp.int32(0))

    keep = (key > mth) | (eqm & (pos <= istar))
    k01 = jnp.where(keep, 1, 0)

    rcum = k01
    sh = 1
    while sh < _RB:
        shifted = jnp.concatenate(
            [jnp.zeros((_NT, sh), jnp.int32), rcum[:, :_RB - sh]], axis=1)
        rcum = rcum + shifted
        sh *= 2
    rtot = rcum[:, _RB - 1:_RB]
    roff = jnp.zeros((_NT, 1), jnp.int32)
    sh = 1
    acc = rtot
    while sh < _NT:
        shifted = jnp.concatenate(
            [jnp.zeros((sh, 1), jnp.int32), acc[:_NT - sh, :]], axis=0)
        acc = acc + shifted
        sh *= 2
    roff = jnp.concatenate(
        [jnp.zeros((1, 1), jnp.int32), acc[:_NT - 1, :]], axis=0)
    gcum = rcum + roff

    kk1 = lax.broadcasted_iota(jnp.int32, (_KP, 1), 0) + 1
    accI = jnp.zeros((_KP, 1), jnp.float32)
    accS = jnp.zeros((_KP, 1), jnp.float32)
    urank = gcum * k01
    for c in range(_NT):
        u_c = urank[c:c + 1, :]
        p_c = ps[c:c + 1, :]
        oh = jnp.where(u_c == kk1, 1.0, 0.0)
        e_c = (lax.broadcasted_iota(jnp.int32, (1, _RB), 1) + c * _RB
               ).astype(jnp.float32)
        accI = accI + jnp.sum(oh * e_c, axis=1, keepdims=True)
        accS = accS + jnp.sum(oh * p_c, axis=1, keepdims=True)

    idx_i = accI.astype(jnp.int32)
    qk = lax.shift_right_arithmetic(idx_i * _M15, _SH15)
    oidx_ref[...] = idx_i.reshape(1, _KP)
    osc_ref[...] = accS.reshape(1, _KP)
    ob_ref[...] = qk.reshape(1, _KP)
    oe_ref[...] = (idx_i - qk * (_L - 1)).reshape(1, _KP)

    sl = sl_ref[...].astype(jnp.float32)
    spl = jnp.minimum(jnp.ceil(_PRUNE_RATIO * sl).astype(jnp.int32), _K)
    ri = lax.broadcasted_iota(jnp.int32, (_K, _K), 0)
    rj = lax.broadcasted_iota(jnp.int32, (_K, _K), 1)
    vi = ri < spl
    vj = rj < spl
    sq = jnp.where(vi & vj, 1.0, 0.0)
    sq_ref[...] = sq
    tri_ref[...] = sq * jnp.where(rj <= ri, 1.0, 0.0)


_CHUNKS = ((0, 104), (104, 104), (208, 104), (312, 104))


def _sc_body(idx_hbm, svf_hbm, fv_hbm, idx_v, idxb_v, rows_v, sem):
    cc = lax.axis_index("c")
    ss = lax.axis_index("s")
    wid = ss * 2 + cc

    @pl.when(wid < _B)
    def _():
        b = wid
        pltpu.sync_copy(idx_hbm.at[b], idx_v)
        for j in range(_KP // 16):
            iv = idx_v[pl.ds(j * 16, 16)]
            idxb_v[pl.ds(j * 16, 16)] = iv + b * _TL

        for (s0, n) in _CHUNKS:
            pltpu.async_copy(svf_hbm.at[idxb_v.at[pl.ds(s0, n)]],
                             rows_v.at[pl.ds(0, n)], sem).wait()
            pltpu.sync_copy(rows_v.at[pl.ds(0, n)],
                            fv_hbm.at[pl.ds(b * _KP + s0, n)])


def _run_score(sv2, sl2, gbm, glm, W1, b1, W2, b2, W3, b3):
    return pl.pallas_call(
        _score_body,
        grid=(_B, _NT),
        in_specs=[
            pl.BlockSpec((None, _RB, _D), lambda b, i: (b, i, 0)),
            pl.BlockSpec((None, 1, 1), lambda b, i: (b, 0, 0)),
            pl.BlockSpec((None, gbm.shape[1], 1), lambda b, i: (b, 0, 0)),
            pl.BlockSpec((None, glm.shape[1], 1), lambda b, i: (b, 0, 0)),
            pl.BlockSpec((_D, _H), lambda b, i: (0, 0)),
            pl.BlockSpec((1, _H), lambda b, i: (0, 0)),
            pl.BlockSpec((_H, _H), lambda b, i: (0, 0)),
            pl.BlockSpec((1, _H), lambda b, i: (0, 0)),
            pl.BlockSpec((1, _H), lambda b, i: (0, 0)),
            pl.BlockSpec((1, 1), lambda b, i: (0, 0)),
        ],
        out_specs=[
            pl.BlockSpec((8, _RB), lambda b, i: ((b * _NT + i) // 8, 0)),
            pl.BlockSpec((1, 1), lambda b, i: (0, 0)),
        ],
        out_shape=[
            jax.ShapeDtypeStruct((_B * _NT, _RB), jnp.float32),
            jax.ShapeDtypeStruct((1, 1), jnp.float32),
        ],
    )(sv2, sl2, gbm, glm, W1, b1, W2, b2, W3, b3)


def _run_topk(ps3, sl3):
    return pl.pallas_call(
        _topk_body,
        grid=(_B,),
        in_specs=[
            pl.BlockSpec((None, _NT, _RB), lambda b: (b, 0, 0)),
            pl.BlockSpec((None, 1, 1), lambda b: (b, 0, 0)),
        ],
        out_specs=[
            pl.BlockSpec((None, 1, _KP), lambda b: (b, 0, 0)),
            pl.BlockSpec((None, 1, _KP), lambda b: (b, 0, 0)),
            pl.BlockSpec((None, 1, _KP), lambda b: (b, 0, 0)),
            pl.BlockSpec((None, 1, _KP), lambda b: (b, 0, 0)),
            pl.BlockSpec((None, _K, _K), lambda b: (b, 0, 0)),
            pl.BlockSpec((None, _K, _K), lambda b: (b, 0, 0)),
        ],
        out_shape=[
            jax.ShapeDtypeStruct((_B, 1, _KP), jnp.int32),
            jax.ShapeDtypeStruct((_B, 1, _KP), jnp.float32),
            jax.ShapeDtypeStruct((_B, 1, _KP), jnp.int32),
            jax.ShapeDtypeStruct((_B, 1, _KP), jnp.int32),
            jax.ShapeDtypeStruct((_B, _K, _K), jnp.float32),
            jax.ShapeDtypeStruct((_B, _K, _K), jnp.float32),
        ],
    )(ps3, sl3)


@functools.cache
def _make_prune():
    return pl.kernel(
        _sc_body,
        mesh=plsc.VectorSubcoreMesh(core_axis_name="c", subcore_axis_name="s"),
        out_type=[
            jax.ShapeDtypeStruct((_B * _KP, _D), jnp.float32),
        ],
        scratch_types=[
            pltpu.VMEM((_KP,), jnp.int32),
            pltpu.VMEM((_KP,), jnp.int32),
            pltpu.VMEM((104, _D), jnp.float32),
            pltpu.SemaphoreType.DMA,
        ],
    )


def _run_prune(idx2, svf):
    return _make_prune()(idx2, svf)


def kernel(span_vecs, span_mask, W1, b1, W2, b2, W3, b3,
           span_begin, span_end, sequence_lengths, gold_spans):
    B, T, L, D = span_vecs.shape
    H = W1.shape[1]
    G = gold_spans.shape[1]
    sv2 = span_vecs.reshape(B, T * L, D)
    sl2 = sequence_lengths.reshape(B, 1, 1)
    gbm = gold_spans[..., 0].reshape(B, G, 1)
    glm = (gold_spans[..., 1] - gold_spans[..., 0]).reshape(B, G, 1)

    ps, loss = _run_score(sv2, sl2, gbm, glm, W1, b1.reshape(1, H),
                          W2, b2.reshape(1, H), W3.reshape(1, H),
                          b3.reshape(1, 1))
    oidx, osc, ob, oe, sq, tri = _run_topk(
        ps.reshape(B, _NT, _RB), sl2)
    idx2 = oidx.reshape(B, _KP)
    fv, = _run_prune(idx2, sv2.reshape(B * T * L, D))

    obj = loss[0, 0]
    prune_scores = ps.reshape(B, T, L, 1)
    f_vecs = fv.reshape(B, _KP, D)[:, :_K]
    f_scores = osc.reshape(B, _KP)[:, :_K, None]
    f_begin = ob.reshape(B, _KP)[:, :_K]
    f_end = oe.reshape(B, _KP)[:, :_K]
    idx = idx2[:, :_K]
    return (obj, prune_scores, f_vecs, f_scores, f_begin, f_end, sq, tri, idx)


# speedup vs baseline: 1.5934x; 1.2819x over previous
import functools

import jax
import jax.numpy as jnp
from jax import lax
from jax.experimental import pallas as pl
from jax.experimental.pallas import tpu as pltpu
from jax.experimental.pallas import tpu_sc as plsc

_B, _T, _L, _D, _H = 4, 2048, 15, 256, 128
_TL = _T * _L
_K = 410
_KP = 416
_PRUNE_RATIO = 0.2
_TT = 128
_RB = _TT * _L
_NT = _T // _TT
_CH = 1920
_NCH = _TL // _CH
_M15 = 69906
_SH15 = 20


def _score_body(x_ref, sl_ref, gb_ref, gl_ref, w1_ref, b1_ref, w2_ref, b2_ref,
                w3_ref, b3_ref, ps_ref, loss_ref):
    b = pl.program_id(0)
    i = pl.program_id(1)
    x = x_ref[...].reshape(_RB, _D)
    h = jnp.maximum(jnp.dot(x, w1_ref[...]) + b1_ref[...], 0.0)
    h = jnp.maximum(jnp.dot(h, w2_ref[...]) + b2_ref[...], 0.0)
    sr = lax.dot_general(w3_ref[...], h, (((1,), (1,)), ((), ())))
    sr = sr + b3_ref[0, 0]

    r = lax.broadcasted_iota(jnp.int32, (1, _RB), 1)
    q = lax.shift_right_arithmetic(r * _M15, _SH15)
    t_row = i * _TT + q
    l_row = r - q * _L
    slen = sl_ref[0, 0]
    m = jnp.where(t_row + l_row < slen, 1.0, 0.0)
    ps = sr - (1.0 - m) * 10000.0
    ps_ref[pl.ds(lax.rem(i, 8), 1), :] = ps

    gb = gb_ref[...]
    gl = gl_ref[...]
    validg = (gl >= 0) & (gl < _L) & (gb >= 0) & (gb < _T)
    hit = (t_row == gb) & (l_row == gl) & validg
    tgt = jnp.any(hit, axis=0, keepdims=True).astype(jnp.float32)

    bce = jnp.maximum(ps, 0.0) - ps * tgt + jnp.log(1.0 + jnp.exp(-jnp.abs(ps)))
    part = jnp.sum(bce * m)

    @pl.when((b == 0) & (i == 0))
    def _():
        loss_ref[...] = jnp.zeros((1, 1), jnp.float32)

    loss_ref[...] += part


def _topk_body(ps_ref, sl_ref, oidx_ref, osc_ref, ob_ref, oe_ref,
               sq_ref, tri_ref):
    ps = ps_ref[...]
    bits = lax.bitcast_convert_type(ps, jnp.int32)
    key = jnp.where(bits >= 0, bits, bits ^ 0x7FFFFFFF)

    def cnt_ge(c):
        return jnp.sum(jnp.where(key >= c, 1, 0))

    def step(j, mcur):
        bit = 30 - j
        cand = mcur + lax.shift_left(jnp.int32(1), bit)
        return jnp.where(cnt_ge(cand) >= _K, cand, mcur)

    m0 = jnp.where(cnt_ge(jnp.int32(0)) >= _K,
                   jnp.int32(0), jnp.int32(-2147483648))
    mth = lax.fori_loop(0, 31, step, m0)
    need = _K - jnp.sum(jnp.where(key > mth, 1, 0))

    eqm = key == mth
    pos = (lax.broadcasted_iota(jnp.int32, (_NT, _RB), 0) * _RB
           + lax.broadcasted_iota(jnp.int32, (_NT, _RB), 1))

    def step2(j, x):
        bit = 14 - j
        xc = x + lax.shift_left(jnp.int32(1), bit)
        cnt = jnp.sum(jnp.where(eqm & (pos < xc), 1, 0))
        return jnp.where(cnt < need, xc, x)

    istar = lax.fori_loop(0, 15, step2, jnp.int32(0))

    keep = (key > mth) | (eqm & (pos <= istar))
    k01 = jnp.where(keep, 1, 0)

    rcum = k01
    sh = 1
    while sh < _RB:
        shifted = jnp.concatenate(
            [jnp.zeros((_NT, sh), jnp.int32), rcum[:, :_RB - sh]], axis=1)
        rcum = rcum + shifted
        sh *= 2
    rtot = rcum[:, _RB - 1:_RB]
    roff = jnp.zeros((_NT, 1), jnp.int32)
    sh = 1
    acc = rtot
    while sh < _NT:
        shifted = jnp.concatenate(
            [jnp.zeros((sh, 1), jnp.int32), acc[:_NT - sh, :]], axis=0)
        acc = acc + shifted
        sh *= 2
    roff = jnp.concatenate(
        [jnp.zeros((1, 1), jnp.int32), acc[:_NT - 1, :]], axis=0)
    gcum = rcum + roff

    kk1 = lax.broadcasted_iota(jnp.int32, (_KP, 1), 0) + 1
    accI = jnp.zeros((_KP, 1), jnp.float32)
    accS = jnp.zeros((_KP, 1), jnp.float32)
    urank = gcum * k01
    for c in range(_NT):
        u_c = urank[c:c + 1, :]
        p_c = ps[c:c + 1, :]
        oh = jnp.where(u_c == kk1, 1.0, 0.0)
        e_c = (lax.broadcasted_iota(jnp.int32, (1, _RB), 1) + c * _RB
               ).astype(jnp.float32)
        accI = accI + jnp.sum(oh * e_c, axis=1, keepdims=True)
        accS = accS + jnp.sum(oh * p_c, axis=1, keepdims=True)

    idx_i = accI.astype(jnp.int32)
    qk = lax.shift_right_arithmetic(idx_i * _M15, _SH15)
    oidx_ref[...] = idx_i.reshape(1, _KP)
    osc_ref[...] = accS.reshape(1, _KP)
    ob_ref[...] = qk.reshape(1, _KP)
    oe_ref[...] = (idx_i - qk * (_L - 1)).reshape(1, _KP)

    sl = sl_ref[...].astype(jnp.float32)
    spl = jnp.minimum(jnp.ceil(_PRUNE_RATIO * sl).astype(jnp.int32), _K)
    ri = lax.broadcasted_iota(jnp.int32, (_K, _K), 0)
    rj = lax.broadcasted_iota(jnp.int32, (_K, _K), 1)
    vi = ri < spl
    vj = rj < spl
    sq = jnp.where(vi & vj, 1.0, 0.0)
    sq_ref[...] = sq
    tri_ref[...] = sq * jnp.where(rj <= ri, 1.0, 0.0)


_CHUNKS = ((0, 104), (104, 104), (208, 104), (312, 104))


def _sc_body(idx_hbm, svf_hbm, fv_hbm, idx_v, idxb_v, rows_v, sem):
    cc = lax.axis_index("c")
    ss = lax.axis_index("s")
    wid = ss * 2 + cc

    @pl.when(wid < _B)
    def _():
        b = wid
        pltpu.sync_copy(idx_hbm.at[b], idx_v)
        for j in range(_KP // 16):
            iv = idx_v[pl.ds(j * 16, 16)]
            idxb_v[pl.ds(j * 16, 16)] = iv + b * _TL

        for (s0, n) in _CHUNKS:
            pltpu.async_copy(svf_hbm.at[idxb_v.at[pl.ds(s0, n)]],
                             rows_v.at[pl.ds(0, n)], sem).wait()
            pltpu.sync_copy(rows_v.at[pl.ds(0, n)],
                            fv_hbm.at[pl.ds(b * _KP + s0, n)])


def _run_score(sv4, sl2, gbm, glm, W1, b1, W2, b2, W3, b3):
    return pl.pallas_call(
        _score_body,
        grid=(_B, _NT),
        in_specs=[
            pl.BlockSpec((None, _TT, _L, _D), lambda b, i: (b, i, 0, 0)),
            pl.BlockSpec((None, 1, 1), lambda b, i: (b, 0, 0)),
            pl.BlockSpec((None, gbm.shape[1], 1), lambda b, i: (b, 0, 0)),
            pl.BlockSpec((None, glm.shape[1], 1), lambda b, i: (b, 0, 0)),
            pl.BlockSpec((_D, _H), lambda b, i: (0, 0)),
            pl.BlockSpec((1, _H), lambda b, i: (0, 0)),
            pl.BlockSpec((_H, _H), lambda b, i: (0, 0)),
            pl.BlockSpec((1, _H), lambda b, i: (0, 0)),
            pl.BlockSpec((1, _H), lambda b, i: (0, 0)),
            pl.BlockSpec((1, 1), lambda b, i: (0, 0)),
        ],
        out_specs=[
            pl.BlockSpec((8, _RB), lambda b, i: ((b * _NT + i) // 8, 0)),
            pl.BlockSpec((1, 1), lambda b, i: (0, 0)),
        ],
        out_shape=[
            jax.ShapeDtypeStruct((_B * _NT, _RB), jnp.float32),
            jax.ShapeDtypeStruct((1, 1), jnp.float32),
        ],
    )(sv4, sl2, gbm, glm, W1, b1, W2, b2, W3, b3)


def _run_topk(ps3, sl3):
    return pl.pallas_call(
        _topk_body,
        grid=(_B,),
        in_specs=[
            pl.BlockSpec((None, _NT, _RB), lambda b: (b, 0, 0)),
            pl.BlockSpec((None, 1, 1), lambda b: (b, 0, 0)),
        ],
        out_specs=[
            pl.BlockSpec((None, 1, _KP), lambda b: (b, 0, 0)),
            pl.BlockSpec((None, 1, _KP), lambda b: (b, 0, 0)),
            pl.BlockSpec((None, 1, _KP), lambda b: (b, 0, 0)),
            pl.BlockSpec((None, 1, _KP), lambda b: (b, 0, 0)),
            pl.BlockSpec((None, _K, _K), lambda b: (b, 0, 0)),
            pl.BlockSpec((None, _K, _K), lambda b: (b, 0, 0)),
        ],
        out_shape=[
            jax.ShapeDtypeStruct((_B, 1, _KP), jnp.int32),
            jax.ShapeDtypeStruct((_B, 1, _KP), jnp.float32),
            jax.ShapeDtypeStruct((_B, 1, _KP), jnp.int32),
            jax.ShapeDtypeStruct((_B, 1, _KP), jnp.int32),
            jax.ShapeDtypeStruct((_B, _K, _K), jnp.float32),
            jax.ShapeDtypeStruct((_B, _K, _K), jnp.float32),
        ],
    )(ps3, sl3)


@functools.cache
def _make_prune():
    return pl.kernel(
        _sc_body,
        mesh=plsc.VectorSubcoreMesh(core_axis_name="c", subcore_axis_name="s"),
        out_type=[
            jax.ShapeDtypeStruct((_B * _KP, _D), jnp.float32),
        ],
        scratch_types=[
            pltpu.VMEM((_KP,), jnp.int32),
            pltpu.VMEM((_KP,), jnp.int32),
            pltpu.VMEM((104, _D), jnp.float32),
            pltpu.SemaphoreType.DMA,
        ],
    )


def _run_prune(idx2, svf):
    return _make_prune()(idx2, svf)


def kernel(span_vecs, span_mask, W1, b1, W2, b2, W3, b3,
           span_begin, span_end, sequence_lengths, gold_spans):
    B, T, L, D = span_vecs.shape
    H = W1.shape[1]
    G = gold_spans.shape[1]
    sv2 = span_vecs.reshape(B, T * L, D)
    sl2 = sequence_lengths.reshape(B, 1, 1)
    gbm = gold_spans[..., 0].reshape(B, G, 1)
    glm = (gold_spans[..., 1] - gold_spans[..., 0]).reshape(B, G, 1)

    ps, loss = _run_score(span_vecs, sl2, gbm, glm, W1, b1.reshape(1, H),
                          W2, b2.reshape(1, H), W3.reshape(1, H),
                          b3.reshape(1, 1))
    oidx, osc, ob, oe, sq, tri = _run_topk(
        ps.reshape(B, _NT, _RB), sl2)
    idx2 = oidx.reshape(B, _KP)
    fv, = _run_prune(idx2, sv2.reshape(B * T * L, D))

    obj = loss[0, 0]
    prune_scores = ps.reshape(B, T, L, 1)
    f_vecs = fv.reshape(B, _KP, D)[:, :_K]
    f_scores = osc.reshape(B, _KP)[:, :_K, None]
    f_begin = ob.reshape(B, _KP)[:, :_K]
    f_end = oe.reshape(B, _KP)[:, :_K]
    idx = idx2[:, :_K]
    return (obj, prune_scores, f_vecs, f_scores, f_begin, f_end, sq, tri, idx)


# speedup vs baseline: 1.6468x; 1.0335x over previous
import functools

import jax
import jax.numpy as jnp
from jax import lax
from jax.experimental import pallas as pl
from jax.experimental.pallas import tpu as pltpu
from jax.experimental.pallas import tpu_sc as plsc

_B, _T, _L, _D, _H = 4, 2048, 15, 256, 128
_TL = _T * _L
_K = 410
_KP = 416
_PRUNE_RATIO = 0.2
_TT = 256
_RB = _TT * _L
_NT = _T // _TT
_CH = 1920
_NCH = _TL // _CH
_M15 = 69906
_SH15 = 20


def _score_body(x_ref, sl_ref, gb_ref, gl_ref, w1_ref, b1_ref, w2_ref, b2_ref,
                w3_ref, b3_ref, ps_ref, loss_ref):
    b = pl.program_id(0)
    i = pl.program_id(1)
    x = x_ref[...].reshape(_RB, _D)
    h = jnp.maximum(jnp.dot(x, w1_ref[...]) + b1_ref[...], 0.0)
    h = jnp.maximum(jnp.dot(h, w2_ref[...]) + b2_ref[...], 0.0)
    sr = lax.dot_general(w3_ref[...], h, (((1,), (1,)), ((), ())))
    sr = sr + b3_ref[0, 0]

    r = lax.broadcasted_iota(jnp.int32, (1, _RB), 1)
    q = lax.shift_right_arithmetic(r * _M15, _SH15)
    t_row = i * _TT + q
    l_row = r - q * _L
    slen = sl_ref[0, 0]
    m = jnp.where(t_row + l_row < slen, 1.0, 0.0)
    ps = sr - (1.0 - m) * 10000.0
    ps_ref[pl.ds(lax.rem(i, 8), 1), :] = ps

    gb = gb_ref[...]
    gl = gl_ref[...]
    validg = (gl >= 0) & (gl < _L) & (gb >= 0) & (gb < _T)
    hit = (t_row == gb) & (l_row == gl) & validg
    tgt = jnp.any(hit, axis=0, keepdims=True).astype(jnp.float32)

    bce = jnp.maximum(ps, 0.0) - ps * tgt + jnp.log(1.0 + jnp.exp(-jnp.abs(ps)))
    part = jnp.sum(bce * m)

    @pl.when((b == 0) & (i == 0))
    def _():
        loss_ref[...] = jnp.zeros((1, 1), jnp.float32)

    loss_ref[...] += part


def _topk_body(ps_ref, sl_ref, oidx_ref, osc_ref, ob_ref, oe_ref,
               sq_ref, tri_ref):
    ps = ps_ref[...]
    bits = lax.bitcast_convert_type(ps, jnp.int32)
    key = jnp.where(bits >= 0, bits, bits ^ 0x7FFFFFFF)

    def cnt_ge(c):
        return jnp.sum(jnp.where(key >= c, 1, 0))

    def step(j, mcur):
        bit = 30 - j
        cand = mcur + lax.shift_left(jnp.int32(1), bit)
        return jnp.where(cnt_ge(cand) >= _K, cand, mcur)

    m0 = jnp.where(cnt_ge(jnp.int32(0)) >= _K,
                   jnp.int32(0), jnp.int32(-2147483648))
    mth = lax.fori_loop(0, 31, step, m0)
    need = _K - jnp.sum(jnp.where(key > mth, 1, 0))

    eqm = key == mth
    pos = (lax.broadcasted_iota(jnp.int32, (_NT, _RB), 0) * _RB
           + lax.broadcasted_iota(jnp.int32, (_NT, _RB), 1))

    def step2(j, x):
        bit = 14 - j
        xc = x + lax.shift_left(jnp.int32(1), bit)
        cnt = jnp.sum(jnp.where(eqm & (pos < xc), 1, 0))
        return jnp.where(cnt < need, xc, x)

    istar = lax.fori_loop(0, 15, step2, jnp.int32(0))

    keep = (key > mth) | (eqm & (pos <= istar))
    k01 = jnp.where(keep, 1, 0)

    rcum = k01
    sh = 1
    while sh < _RB:
        shifted = jnp.concatenate(
            [jnp.zeros((_NT, sh), jnp.int32), rcum[:, :_RB - sh]], axis=1)
        rcum = rcum + shifted
        sh *= 2
    rtot = rcum[:, _RB - 1:_RB]
    roff = jnp.zeros((_NT, 1), jnp.int32)
    sh = 1
    acc = rtot
    while sh < _NT:
        shifted = jnp.concatenate(
            [jnp.zeros((sh, 1), jnp.int32), acc[:_NT - sh, :]], axis=0)
        acc = acc + shifted
        sh *= 2
    roff = jnp.concatenate(
        [jnp.zeros((1, 1), jnp.int32), acc[:_NT - 1, :]], axis=0)
    gcum = rcum + roff

    kk1 = lax.broadcasted_iota(jnp.int32, (_KP, 1), 0) + 1
    accI = jnp.zeros((_KP, 1), jnp.float32)
    accS = jnp.zeros((_KP, 1), jnp.float32)
    urank = gcum * k01
    for c in range(_NT):
        u_c = urank[c:c + 1, :]
        p_c = ps[c:c + 1, :]
        oh = jnp.where(u_c == kk1, 1.0, 0.0)
        e_c = (lax.broadcasted_iota(jnp.int32, (1, _RB), 1) + c * _RB
               ).astype(jnp.float32)
        accI = accI + jnp.sum(oh * e_c, axis=1, keepdims=True)
        accS = accS + jnp.sum(oh * p_c, axis=1, keepdims=True)

    idx_i = accI.astype(jnp.int32)
    qk = lax.shift_right_arithmetic(idx_i * _M15, _SH15)
    oidx_ref[...] = idx_i.reshape(1, _KP)
    osc_ref[...] = accS.reshape(1, _KP)
    ob_ref[...] = qk.reshape(1, _KP)
    oe_ref[...] = (idx_i - qk * (_L - 1)).reshape(1, _KP)

    sl = sl_ref[...].astype(jnp.float32)
    spl = jnp.minimum(jnp.ceil(_PRUNE_RATIO * sl).astype(jnp.int32), _K)
    ri = lax.broadcasted_iota(jnp.int32, (_K, _K), 0)
    rj = lax.broadcasted_iota(jnp.int32, (_K, _K), 1)
    vi = ri < spl
    vj = rj < spl
    sq = jnp.where(vi & vj, 1.0, 0.0)
    sq_ref[...] = sq
    tri_ref[...] = sq * jnp.where(rj <= ri, 1.0, 0.0)


_CHUNKS = ((0, 104), (104, 104), (208, 104), (312, 104))


def _sc_body(idx_hbm, svf_hbm, fv_hbm, idx_v, idxb_v, rows_v, sem):
    cc = lax.axis_index("c")
    ss = lax.axis_index("s")
    wid = ss * 2 + cc

    @pl.when(wid < _B)
    def _():
        b = wid
        pltpu.sync_copy(idx_hbm.at[b], idx_v)
        for j in range(_KP // 16):
            iv = idx_v[pl.ds(j * 16, 16)]
            idxb_v[pl.ds(j * 16, 16)] = iv + b * _TL

        for (s0, n) in _CHUNKS:
            pltpu.async_copy(svf_hbm.at[idxb_v.at[pl.ds(s0, n)]],
                             rows_v.at[pl.ds(0, n)], sem).wait()
            pltpu.sync_copy(rows_v.at[pl.ds(0, n)],
                            fv_hbm.at[pl.ds(b * _KP + s0, n)])


def _run_score(sv4, sl2, gbm, glm, W1, b1, W2, b2, W3, b3):
    return pl.pallas_call(
        _score_body,
        grid=(_B, _NT),
        in_specs=[
            pl.BlockSpec((None, _TT, _L, _D), lambda b, i: (b, i, 0, 0)),
            pl.BlockSpec((None, 1, 1), lambda b, i: (b, 0, 0)),
            pl.BlockSpec((None, gbm.shape[1], 1), lambda b, i: (b, 0, 0)),
            pl.BlockSpec((None, glm.shape[1], 1), lambda b, i: (b, 0, 0)),
            pl.BlockSpec((_D, _H), lambda b, i: (0, 0)),
            pl.BlockSpec((1, _H), lambda b, i: (0, 0)),
            pl.BlockSpec((_H, _H), lambda b, i: (0, 0)),
            pl.BlockSpec((1, _H), lambda b, i: (0, 0)),
            pl.BlockSpec((1, _H), lambda b, i: (0, 0)),
            pl.BlockSpec((1, 1), lambda b, i: (0, 0)),
        ],
        out_specs=[
            pl.BlockSpec((8, _RB), lambda b, i: ((b * _NT + i) // 8, 0)),
            pl.BlockSpec((1, 1), lambda b, i: (0, 0)),
        ],
        out_shape=[
            jax.ShapeDtypeStruct((_B * _NT, _RB), jnp.float32),
            jax.ShapeDtypeStruct((1, 1), jnp.float32),
        ],
    )(sv4, sl2, gbm, glm, W1, b1, W2, b2, W3, b3)


def _run_topk(ps3, sl3):
    return pl.pallas_call(
        _topk_body,
        grid=(_B,),
        in_specs=[
            pl.BlockSpec((None, _NT, _RB), lambda b: (b, 0, 0)),
            pl.BlockSpec((None, 1, 1), lambda b: (b, 0, 0)),
        ],
        out_specs=[
            pl.BlockSpec((None, 1, _KP), lambda b: (b, 0, 0)),
            pl.BlockSpec((None, 1, _KP), lambda b: (b, 0, 0)),
            pl.BlockSpec((None, 1, _KP), lambda b: (b, 0, 0)),
            pl.BlockSpec((None, 1, _KP), lambda b: (b, 0, 0)),
            pl.BlockSpec((None, _K, _K), lambda b: (b, 0, 0)),
            pl.BlockSpec((None, _K, _K), lambda b: (b, 0, 0)),
        ],
        out_shape=[
            jax.ShapeDtypeStruct((_B, 1, _KP), jnp.int32),
            jax.ShapeDtypeStruct((_B, 1, _KP), jnp.float32),
            jax.ShapeDtypeStruct((_B, 1, _KP), jnp.int32),
            jax.ShapeDtypeStruct((_B, 1, _KP), jnp.int32),
            jax.ShapeDtypeStruct((_B, _K, _K), jnp.float32),
            jax.ShapeDtypeStruct((_B, _K, _K), jnp.float32),
        ],
    )(ps3, sl3)


@functools.cache
def _make_prune():
    return pl.kernel(
        _sc_body,
        mesh=plsc.VectorSubcoreMesh(core_axis_name="c", subcore_axis_name="s"),
        out_type=[
            jax.ShapeDtypeStruct((_B * _KP, _D), jnp.float32),
        ],
        scratch_types=[
            pltpu.VMEM((_KP,), jnp.int32),
            pltpu.VMEM((_KP,), jnp.int32),
            pltpu.VMEM((104, _D), jnp.float32),
            pltpu.SemaphoreType.DMA,
        ],
    )


def _run_prune(idx2, svf):
    return _make_prune()(idx2, svf)


def kernel(span_vecs, span_mask, W1, b1, W2, b2, W3, b3,
           span_begin, span_end, sequence_lengths, gold_spans):
    B, T, L, D = span_vecs.shape
    H = W1.shape[1]
    G = gold_spans.shape[1]
    sv2 = span_vecs.reshape(B, T * L, D)
    sl2 = sequence_lengths.reshape(B, 1, 1)
    gbm = gold_spans[..., 0].reshape(B, G, 1)
    glm = (gold_spans[..., 1] - gold_spans[..., 0]).reshape(B, G, 1)

    ps, loss = _run_score(span_vecs, sl2, gbm, glm, W1, b1.reshape(1, H),
                          W2, b2.reshape(1, H), W3.reshape(1, H),
                          b3.reshape(1, 1))
    oidx, osc, ob, oe, sq, tri = _run_topk(
        ps.reshape(B, _NT, _RB), sl2)
    idx2 = oidx.reshape(B, _KP)
    fv, = _run_prune(idx2, sv2.reshape(B * T * L, D))

    obj = loss[0, 0]
    prune_scores = ps.reshape(B, T, L, 1)
    f_vecs = fv.reshape(B, _KP, D)[:, :_K]
    f_scores = osc.reshape(B, _KP)[:, :_K, None]
    f_begin = ob.reshape(B, _KP)[:, :_K]
    f_end = oe.reshape(B, _KP)[:, :_K]
    idx = idx2[:, :_K]
    return (obj, prune_scores, f_vecs, f_scores, f_begin, f_end, sq, tri, idx)
